# initial kernel scaffold (unmeasured)
import jax
import jax.numpy as jnp
from jax import lax
from jax.experimental import pallas as pl
from jax.experimental.pallas import tpu as pltpu

N_DEV = 4
SQ = 2048
HQ = 8
DH = 128
D = HQ * DH
QB = SQ // N_DEV
WIN = 128
NK = QB + 2 * WIN
SCALE = 0.08838834764831843
NEG = -1e9


def kernel(x, Wq, K_ext, V_ext, Wo):
    def body(x_ref, wq_ref, k_ref, v_ref, wo_ref, out_ref,
             xloc, kvb, kvw, xsem, ksend, krecv, agsend, agrecv):
        my = lax.axis_index("i")
        right = (my + 1) % N_DEV

        xcopy = pltpu.make_async_copy(
            x_ref.at[0, pl.ds(my * QB, QB), :], xloc, xsem)
        xcopy.start()

        kvb[0] = k_ref[0].astype(jnp.bfloat16)
        kvb[1] = v_ref[0].astype(jnp.bfloat16)

        bar = pltpu.get_barrier_semaphore()
        for off in range(1, N_DEV):
            pl.semaphore_signal(
                bar, inc=1, device_id=((my + off) % N_DEV,),
                device_id_type=pl.DeviceIdType.MESH)
        pl.semaphore_wait(bar, N_DEV - 1)

        @pl.when(my == 0)
        def _():
            for i, tgt in enumerate((1, 2)):
                s = pltpu.make_async_remote_copy(
                    src_ref=kvb.at[:, pl.ds(tgt * QB - WIN, NK)],
                    dst_ref=kvw,
                    send_sem=ksend.at[i], recv_sem=krecv.at[0],
                    device_id=(tgt,), device_id_type=pl.DeviceIdType.MESH)
                s.start()
            s3 = pltpu.make_async_remote_copy(
                src_ref=kvb.at[:, pl.ds(3 * QB - WIN, NK - WIN)],
                dst_ref=kvw.at[:, pl.ds(0, NK - WIN)],
                send_sem=ksend.at[2], recv_sem=krecv.at[0],
                device_id=(3,), device_id_type=pl.DeviceIdType.MESH)
            s3.start()
            kvw[...] = kvb[:, 0:NK]

        @pl.when(my == 1)
        def _():
            h = pltpu.make_async_remote_copy(
                src_ref=kvb.at[:, pl.ds(0, WIN)],
                dst_ref=kvw.at[:, pl.ds(NK - WIN, WIN)],
                send_sem=ksend.at[0], recv_sem=krecv.at[1],
                device_id=(3,), device_id_type=pl.DeviceIdType.MESH)
            h.start()

        @pl.when((my == 1) | (my == 2))
        def _():
            r = pltpu.make_async_remote_copy(
                src_ref=kvw, dst_ref=kvw,
                send_sem=ksend.at[1], recv_sem=krecv.at[0],
                device_id=(0,), device_id_type=pl.DeviceIdType.MESH)
            r.wait_recv()

        @pl.when(my == 3)
        def _():
            r0 = pltpu.make_async_remote_copy(
                src_ref=kvw.at[:, pl.ds(0, NK - WIN)],
                dst_ref=kvw.at[:, pl.ds(0, NK - WIN)],
                send_sem=ksend.at[1], recv_sem=krecv.at[0],
                device_id=(0,), device_id_type=pl.DeviceIdType.MESH)
            r0.wait_recv()
            r1 = pltpu.make_async_remote_copy(
                src_ref=kvw.at[:, pl.ds(NK - WIN, WIN)],
                dst_ref=kvw.at[:, pl.ds(NK - WIN, WIN)],
                send_sem=ksend.at[1], recv_sem=krecv.at[1],
                device_id=(1,), device_id_type=pl.DeviceIdType.MESH)
            r1.wait_recv()

        xcopy.wait()
        xb = xloc[...].astype(jnp.bfloat16)
        q = jnp.dot(xb, wq_ref[...].astype(jnp.bfloat16),
                    preferred_element_type=jnp.float32)

        base = jnp.maximum(my * QB - WIN, 0)
        qi = my * QB + lax.broadcasted_iota(jnp.int32, (QB, NK), 0)
        kj = base + lax.broadcasted_iota(jnp.int32, (QB, NK), 1)
        band = jnp.abs(qi - kj) <= WIN

        ctxs = []
        for h in range(HQ):
            qh = q[:, h * DH:(h + 1) * DH].astype(jnp.bfloat16)
            kh = kvw[0, :, h, :]
            s = lax.dot_general(qh, kh, (((1,), (1,)), ((), ())),
                                preferred_element_type=jnp.float32) * SCALE
            s = jnp.where(band, s, NEG)
            m = jnp.max(s, axis=1, keepdims=True)
            w = jnp.exp(s - m)
            l = jnp.sum(w, axis=1, keepdims=True)
            p = (w / l).astype(jnp.bfloat16)
            vh = kvw[1, :, h, :]
            ctxs.append(lax.dot_general(p, vh, (((1,), (0,)), ((), ())),
                                        preferred_element_type=jnp.float32))
        ctx = jnp.concatenate(ctxs, axis=1)
        ob = jnp.dot(ctx.astype(jnp.bfloat16),
                     wo_ref[...].astype(jnp.bfloat16),
                     preferred_element_type=jnp.float32)
        out_ref[0, pl.ds(my * QB, QB), :] = ob.astype(jnp.bfloat16)

        for hop in range(N_DEV - 1):
            org = (my - hop) % N_DEV
            sl = pl.ds(org * QB, QB)
            rd = pltpu.make_async_remote_copy(
                src_ref=out_ref.at[0, sl, :],
                dst_ref=out_ref.at[0, sl, :],
                send_sem=agsend.at[hop], recv_sem=agrecv.at[hop],
                device_id=(right,), device_id_type=pl.DeviceIdType.MESH)
            rd.start()
            rd.wait()

        @pl.when(my == 0)
        def _():
            for i, tgt in enumerate((1, 2)):
                s = pltpu.make_async_remote_copy(
                    src_ref=kvb.at[:, pl.ds(tgt * QB - WIN, NK)],
                    dst_ref=kvw,
                    send_sem=ksend.at[i], recv_sem=krecv.at[0],
                    device_id=(tgt,), device_id_type=pl.DeviceIdType.MESH)
                s.wait_send()
            s3 = pltpu.make_async_remote_copy(
                src_ref=kvb.at[:, pl.ds(3 * QB - WIN, NK - WIN)],
                dst_ref=kvw.at[:, pl.ds(0, NK - WIN)],
                send_sem=ksend.at[2], recv_sem=krecv.at[0],
                device_id=(3,), device_id_type=pl.DeviceIdType.MESH)
            s3.wait_send()

        @pl.when(my == 1)
        def _():
            h = pltpu.make_async_remote_copy(
                src_ref=kvb.at[:, pl.ds(0, WIN)],
                dst_ref=kvw.at[:, pl.ds(NK - WIN, WIN)],
                send_sem=ksend.at[0], recv_sem=krecv.at[1],
                device_id=(3,), device_id_type=pl.DeviceIdType.MESH)
            h.wait_send()

    return pl.pallas_call(
        body,
        out_shape=jax.ShapeDtypeStruct((1, SQ, D), jnp.bfloat16),
        in_specs=[
            pl.BlockSpec(memory_space=pl.ANY),
            pl.BlockSpec(memory_space=pltpu.VMEM),
            pl.BlockSpec(memory_space=pltpu.VMEM),
            pl.BlockSpec(memory_space=pltpu.VMEM),
            pl.BlockSpec(memory_space=pltpu.VMEM),
        ],
        out_specs=pl.BlockSpec(memory_space=pltpu.VMEM),
        scratch_shapes=[
            pltpu.VMEM((QB, D), jnp.float32),
            pltpu.VMEM((2, 2048, HQ, DH), jnp.bfloat16),
            pltpu.VMEM((2, NK, HQ, DH), jnp.bfloat16),
            pltpu.SemaphoreType.DMA,
            pltpu.SemaphoreType.DMA((3,)),
            pltpu.SemaphoreType.DMA((2,)),
            pltpu.SemaphoreType.DMA((3,)),
            pltpu.SemaphoreType.DMA((3,)),
        ],
        compiler_params=pltpu.CompilerParams(collective_id=0),
    )(x, Wq, K_ext, V_ext, Wo)


# baseline (device time: 132309 ns/iter reference)
import jax
import jax.numpy as jnp
from jax import lax
from jax.experimental import pallas as pl
from jax.experimental.pallas import tpu as pltpu

N_DEV = 4
SQ = 2048
HQ = 8
DH = 128
D = HQ * DH
QB = SQ // N_DEV
WIN = 128
NK = QB + 2 * WIN
SCALE = 0.08838834764831843
NEG = -1e9


def kernel(x, Wq, K_ext, V_ext, Wo):
    def body(x_ref, wq_ref, k_ref, v_ref, wo_ref, out_ref,
             xloc, kvb, kvw, xsem, ksend, krecv, agsend, agrecv):
        my = lax.axis_index("i")
        right = (my + 1) % N_DEV

        xcopy = pltpu.make_async_copy(
            x_ref.at[0, pl.ds(my * QB, QB), :], xloc, xsem)
        xcopy.start()

        kvb[0] = k_ref[0].astype(jnp.bfloat16)
        kvb[1] = v_ref[0].astype(jnp.bfloat16)

        bar = pltpu.get_barrier_semaphore()
        for off in range(1, N_DEV):
            pl.semaphore_signal(
                bar, inc=1, device_id=((my + off) % N_DEV,),
                device_id_type=pl.DeviceIdType.MESH)
        pl.semaphore_wait(bar, N_DEV - 1)

        @pl.when(my == 0)
        def _():
            for i, tgt in enumerate((1, 2)):
                s = pltpu.make_async_remote_copy(
                    src_ref=kvb.at[:, pl.ds(tgt * QB - WIN, NK)],
                    dst_ref=kvw,
                    send_sem=ksend.at[i], recv_sem=krecv.at[0],
                    device_id=(tgt,), device_id_type=pl.DeviceIdType.MESH)
                s.start()
            s3 = pltpu.make_async_remote_copy(
                src_ref=kvb.at[:, pl.ds(3 * QB - WIN, NK - WIN)],
                dst_ref=kvw.at[:, pl.ds(0, NK - WIN)],
                send_sem=ksend.at[2], recv_sem=krecv.at[0],
                device_id=(3,), device_id_type=pl.DeviceIdType.MESH)
            s3.start()
            kvw[...] = kvb[:, 0:NK]

        @pl.when(my == 1)
        def _():
            h = pltpu.make_async_remote_copy(
                src_ref=kvb.at[:, pl.ds(0, WIN)],
                dst_ref=kvw.at[:, pl.ds(NK - WIN, WIN)],
                send_sem=ksend.at[0], recv_sem=krecv.at[1],
                device_id=(3,), device_id_type=pl.DeviceIdType.MESH)
            h.start()

        @pl.when((my == 1) | (my == 2))
        def _():
            r = pltpu.make_async_remote_copy(
                src_ref=kvw, dst_ref=kvw,
                send_sem=ksend.at[1], recv_sem=krecv.at[0],
                device_id=(0,), device_id_type=pl.DeviceIdType.MESH)
            r.wait_recv()

        @pl.when(my == 3)
        def _():
            r0 = pltpu.make_async_remote_copy(
                src_ref=kvw.at[:, pl.ds(0, NK - WIN)],
                dst_ref=kvw.at[:, pl.ds(0, NK - WIN)],
                send_sem=ksend.at[1], recv_sem=krecv.at[0],
                device_id=(0,), device_id_type=pl.DeviceIdType.MESH)
            r0.wait_recv()
            r1 = pltpu.make_async_remote_copy(
                src_ref=kvw.at[:, pl.ds(NK - WIN, WIN)],
                dst_ref=kvw.at[:, pl.ds(NK - WIN, WIN)],
                send_sem=ksend.at[1], recv_sem=krecv.at[1],
                device_id=(1,), device_id_type=pl.DeviceIdType.MESH)
            r1.wait_recv()

        xcopy.wait()
        xb = xloc[...].astype(jnp.bfloat16)
        q = jnp.dot(xb, wq_ref[...].astype(jnp.bfloat16),
                    preferred_element_type=jnp.float32)

        base = jnp.maximum(my * QB - WIN, 0)
        qi = my * QB + lax.broadcasted_iota(jnp.int32, (QB, NK), 0)
        kj = base + lax.broadcasted_iota(jnp.int32, (QB, NK), 1)
        band = jnp.abs(qi - kj) <= WIN

        ctxs = []
        for h in range(HQ):
            qh = q[:, h * DH:(h + 1) * DH].astype(jnp.bfloat16)
            kh = kvw[0, :, h, :]
            s = lax.dot_general(qh, kh, (((1,), (1,)), ((), ())),
                                preferred_element_type=jnp.float32) * SCALE
            s = jnp.where(band, s, NEG)
            m = jnp.max(s, axis=1, keepdims=True)
            w = jnp.exp(s - m)
            l = jnp.sum(w, axis=1, keepdims=True)
            p = (w / l).astype(jnp.bfloat16)
            vh = kvw[1, :, h, :]
            ctxs.append(lax.dot_general(p, vh, (((1,), (0,)), ((), ())),
                                        preferred_element_type=jnp.float32))
        ctx = jnp.concatenate(ctxs, axis=1)
        ob = jnp.dot(ctx.astype(jnp.bfloat16),
                     wo_ref[...].astype(jnp.bfloat16),
                     preferred_element_type=jnp.float32)
        out_ref[0, pl.ds(my * QB, QB), :] = ob.astype(jnp.bfloat16)

        for hop in range(N_DEV - 1):
            org = (my - hop) % N_DEV
            sl = pl.ds(org * QB, QB)
            rd = pltpu.make_async_remote_copy(
                src_ref=out_ref.at[0, sl, :],
                dst_ref=out_ref.at[0, sl, :],
                send_sem=agsend.at[hop], recv_sem=agrecv.at[hop],
                device_id=(right,), device_id_type=pl.DeviceIdType.MESH)
            rd.start()
            rd.wait()

        @pl.when(my == 0)
        def _():
            for i, tgt in enumerate((1, 2)):
                s = pltpu.make_async_remote_copy(
                    src_ref=kvb.at[:, pl.ds(tgt * QB - WIN, NK)],
                    dst_ref=kvw,
                    send_sem=ksend.at[i], recv_sem=krecv.at[0],
                    device_id=(tgt,), device_id_type=pl.DeviceIdType.MESH)
                s.wait_send()
            s3 = pltpu.make_async_remote_copy(
                src_ref=kvb.at[:, pl.ds(3 * QB - WIN, NK - WIN)],
                dst_ref=kvw.at[:, pl.ds(0, NK - WIN)],
                send_sem=ksend.at[2], recv_sem=krecv.at[0],
                device_id=(3,), device_id_type=pl.DeviceIdType.MESH)
            s3.wait_send()

        @pl.when(my == 1)
        def _():
            h = pltpu.make_async_remote_copy(
                src_ref=kvb.at[:, pl.ds(0, WIN)],
                dst_ref=kvw.at[:, pl.ds(NK - WIN, WIN)],
                send_sem=ksend.at[0], recv_sem=krecv.at[1],
                device_id=(3,), device_id_type=pl.DeviceIdType.MESH)
            h.wait_send()

    return pl.pallas_call(
        body,
        out_shape=jax.ShapeDtypeStruct((1, SQ, D), jnp.bfloat16),
        in_specs=[
            pl.BlockSpec(memory_space=pl.ANY),
            pl.BlockSpec(memory_space=pltpu.VMEM),
            pl.BlockSpec(memory_space=pltpu.VMEM),
            pl.BlockSpec(memory_space=pltpu.VMEM),
            pl.BlockSpec(memory_space=pltpu.VMEM),
        ],
        out_specs=pl.BlockSpec(memory_space=pltpu.VMEM),
        scratch_shapes=[
            pltpu.VMEM((QB, D), jnp.float32),
            pltpu.VMEM((2, 2048, HQ, DH), jnp.bfloat16),
            pltpu.VMEM((2, NK, HQ, DH), jnp.bfloat16),
            pltpu.SemaphoreType.DMA,
            pltpu.SemaphoreType.DMA((3,)),
            pltpu.SemaphoreType.DMA((2,)),
            pltpu.SemaphoreType.DMA((3,)),
            pltpu.SemaphoreType.DMA((3,)),
        ],
        compiler_params=pltpu.CompilerParams(
            collective_id=0, vmem_limit_bytes=60 * 1024 * 1024),
    )(x, Wq, K_ext, V_ext, Wo)


# device time: 115151 ns/iter; 1.1490x vs baseline; 1.1490x over previous
import jax
import jax.numpy as jnp
from jax import lax
from jax.experimental import pallas as pl
from jax.experimental.pallas import tpu as pltpu

N_DEV = 4
SQ = 2048
HQ = 8
DH = 128
D = HQ * DH
QB = SQ // N_DEV
WIN = 128
NK = QB + 2 * WIN
SCALE = 0.08838834764831843
NEG = -1e9


def kernel(x, Wq, K_ext, V_ext, Wo):
    def body(x_ref, wq_ref, k_ref, v_ref, wo_ref, out_ref,
             xloc, kvb, kvw, xsem, ksend, krecv, agsend, agrecv):
        my = lax.axis_index("i")
        right = (my + 1) % N_DEV
        left = (my - 1) % N_DEV

        xcopy = pltpu.make_async_copy(
            x_ref.at[0, pl.ds(my * QB, QB), :], xloc, xsem)
        xcopy.start()

        kvb[0] = k_ref[0].astype(jnp.bfloat16)
        kvb[1] = v_ref[0].astype(jnp.bfloat16)

        bar = pltpu.get_barrier_semaphore()
        for off in range(1, N_DEV):
            pl.semaphore_signal(
                bar, inc=1, device_id=((my + off) % N_DEV,),
                device_id_type=pl.DeviceIdType.MESH)
        pl.semaphore_wait(bar, N_DEV - 1)

        @pl.when(my == 0)
        def _():
            for i, tgt in enumerate((1, 2)):
                s = pltpu.make_async_remote_copy(
                    src_ref=kvb.at[:, pl.ds(tgt * QB - WIN, NK)],
                    dst_ref=kvw,
                    send_sem=ksend.at[i], recv_sem=krecv.at[0],
                    device_id=(tgt,), device_id_type=pl.DeviceIdType.MESH)
                s.start()
            s3 = pltpu.make_async_remote_copy(
                src_ref=kvb.at[:, pl.ds(3 * QB - WIN, NK - WIN)],
                dst_ref=kvw.at[:, pl.ds(0, NK - WIN)],
                send_sem=ksend.at[2], recv_sem=krecv.at[0],
                device_id=(3,), device_id_type=pl.DeviceIdType.MESH)
            s3.start()
            kvw[...] = kvb[:, 0:NK]

        @pl.when(my == 1)
        def _():
            h = pltpu.make_async_remote_copy(
                src_ref=kvb.at[:, pl.ds(0, WIN)],
                dst_ref=kvw.at[:, pl.ds(NK - WIN, WIN)],
                send_sem=ksend.at[0], recv_sem=krecv.at[1],
                device_id=(3,), device_id_type=pl.DeviceIdType.MESH)
            h.start()

        @pl.when((my == 1) | (my == 2))
        def _():
            r = pltpu.make_async_remote_copy(
                src_ref=kvw, dst_ref=kvw,
                send_sem=ksend.at[1], recv_sem=krecv.at[0],
                device_id=(0,), device_id_type=pl.DeviceIdType.MESH)
            r.wait_recv()

        @pl.when(my == 3)
        def _():
            r0 = pltpu.make_async_remote_copy(
                src_ref=kvw.at[:, pl.ds(0, NK - WIN)],
                dst_ref=kvw.at[:, pl.ds(0, NK - WIN)],
                send_sem=ksend.at[1], recv_sem=krecv.at[0],
                device_id=(0,), device_id_type=pl.DeviceIdType.MESH)
            r0.wait_recv()
            r1 = pltpu.make_async_remote_copy(
                src_ref=kvw.at[:, pl.ds(NK - WIN, WIN)],
                dst_ref=kvw.at[:, pl.ds(NK - WIN, WIN)],
                send_sem=ksend.at[1], recv_sem=krecv.at[1],
                device_id=(1,), device_id_type=pl.DeviceIdType.MESH)
            r1.wait_recv()

        xcopy.wait()
        xb = xloc[...].astype(jnp.bfloat16)
        q = jnp.dot(xb, wq_ref[...].astype(jnp.bfloat16),
                    preferred_element_type=jnp.float32)

        base = jnp.maximum(my * QB - WIN, 0)
        qi = my * QB + lax.broadcasted_iota(jnp.int32, (QB, NK), 0)
        kj = base + lax.broadcasted_iota(jnp.int32, (QB, NK), 1)
        band = jnp.abs(qi - kj) <= WIN

        ctxs = []
        for h in range(HQ):
            qh = q[:, h * DH:(h + 1) * DH].astype(jnp.bfloat16)
            kh = kvw[0, :, h, :]
            s = lax.dot_general(qh, kh, (((1,), (1,)), ((), ())),
                                preferred_element_type=jnp.float32) * SCALE
            s = jnp.where(band, s, NEG)
            m = jnp.max(s, axis=1, keepdims=True)
            w = jnp.exp(s - m)
            l = jnp.sum(w, axis=1, keepdims=True)
            p = (w / l).astype(jnp.bfloat16)
            vh = kvw[1, :, h, :]
            ctxs.append(lax.dot_general(p, vh, (((1,), (0,)), ((), ())),
                                        preferred_element_type=jnp.float32))
        ctx = jnp.concatenate(ctxs, axis=1)
        ob = jnp.dot(ctx.astype(jnp.bfloat16),
                     wo_ref[...].astype(jnp.bfloat16),
                     preferred_element_type=jnp.float32)
        out_ref[0, pl.ds(my * QB, QB), :] = ob.astype(jnp.bfloat16)

        my_sl = pl.ds(my * QB, QB)
        s_l = pltpu.make_async_remote_copy(
            src_ref=out_ref.at[0, my_sl, :], dst_ref=out_ref.at[0, my_sl, :],
            send_sem=agsend.at[0], recv_sem=agrecv.at[0],
            device_id=(left,), device_id_type=pl.DeviceIdType.MESH)
        s_r = pltpu.make_async_remote_copy(
            src_ref=out_ref.at[0, my_sl, :], dst_ref=out_ref.at[0, my_sl, :],
            send_sem=agsend.at[1], recv_sem=agrecv.at[1],
            device_id=(right,), device_id_type=pl.DeviceIdType.MESH)
        s_l.start()
        s_r.start()
        s_l.wait()
        s_r.wait()
        f_r = pltpu.make_async_remote_copy(
            src_ref=out_ref.at[0, pl.ds(left * QB, QB // 2), :],
            dst_ref=out_ref.at[0, pl.ds(left * QB, QB // 2), :],
            send_sem=agsend.at[2], recv_sem=agrecv.at[2],
            device_id=(right,), device_id_type=pl.DeviceIdType.MESH)
        f_l = pltpu.make_async_remote_copy(
            src_ref=out_ref.at[0, pl.ds(right * QB + QB // 2, QB // 2), :],
            dst_ref=out_ref.at[0, pl.ds(right * QB + QB // 2, QB // 2), :],
            send_sem=agsend.at[3], recv_sem=agrecv.at[3],
            device_id=(left,), device_id_type=pl.DeviceIdType.MESH)
        f_r.start()
        f_l.start()
        f_r.wait()
        f_l.wait()

        @pl.when(my == 0)
        def _():
            for i, tgt in enumerate((1, 2)):
                s = pltpu.make_async_remote_copy(
                    src_ref=kvb.at[:, pl.ds(tgt * QB - WIN, NK)],
                    dst_ref=kvw,
                    send_sem=ksend.at[i], recv_sem=krecv.at[0],
                    device_id=(tgt,), device_id_type=pl.DeviceIdType.MESH)
                s.wait_send()
            s3 = pltpu.make_async_remote_copy(
                src_ref=kvb.at[:, pl.ds(3 * QB - WIN, NK - WIN)],
                dst_ref=kvw.at[:, pl.ds(0, NK - WIN)],
                send_sem=ksend.at[2], recv_sem=krecv.at[0],
                device_id=(3,), device_id_type=pl.DeviceIdType.MESH)
            s3.wait_send()

        @pl.when(my == 1)
        def _():
            h = pltpu.make_async_remote_copy(
                src_ref=kvb.at[:, pl.ds(0, WIN)],
                dst_ref=kvw.at[:, pl.ds(NK - WIN, WIN)],
                send_sem=ksend.at[0], recv_sem=krecv.at[1],
                device_id=(3,), device_id_type=pl.DeviceIdType.MESH)
            h.wait_send()

    return pl.pallas_call(
        body,
        out_shape=jax.ShapeDtypeStruct((1, SQ, D), jnp.bfloat16),
        in_specs=[
            pl.BlockSpec(memory_space=pl.ANY),
            pl.BlockSpec(memory_space=pltpu.VMEM),
            pl.BlockSpec(memory_space=pltpu.VMEM),
            pl.BlockSpec(memory_space=pltpu.VMEM),
            pl.BlockSpec(memory_space=pltpu.VMEM),
        ],
        out_specs=pl.BlockSpec(memory_space=pltpu.VMEM),
        scratch_shapes=[
            pltpu.VMEM((QB, D), jnp.float32),
            pltpu.VMEM((2, 2048, HQ, DH), jnp.bfloat16),
            pltpu.VMEM((2, NK, HQ, DH), jnp.bfloat16),
            pltpu.SemaphoreType.DMA,
            pltpu.SemaphoreType.DMA((3,)),
            pltpu.SemaphoreType.DMA((2,)),
            pltpu.SemaphoreType.DMA((4,)),
            pltpu.SemaphoreType.DMA((4,)),
        ],
        compiler_params=pltpu.CompilerParams(
            collective_id=0, vmem_limit_bytes=60 * 1024 * 1024),
    )(x, Wq, K_ext, V_ext, Wo)


# device time: 91696 ns/iter; 1.4429x vs baseline; 1.2558x over previous
import jax
import jax.numpy as jnp
from jax import lax
from jax.experimental import pallas as pl
from jax.experimental.pallas import tpu as pltpu

N_DEV = 4
SQ = 2048
HQ = 8
DH = 128
D = HQ * DH
QB = SQ // N_DEV
WIN = 128
NK = QB + 2 * WIN
SCALE = 0.08838834764831843
NEG = -1e9


def kernel(x, Wq, K_ext, V_ext, Wo):
    def body(x_ref, wq_ref, k_ref, v_ref, wo_ref, out_ref,
             xloc, kvb, kvw, xsem, ksend, krecv, agsend, agrecv):
        my = lax.axis_index("i")
        right = (my + 1) % N_DEV
        left = (my - 1) % N_DEV

        xcopy = pltpu.make_async_copy(
            x_ref.at[0, pl.ds(my * QB, QB), :], xloc, xsem)
        xcopy.start()

        @pl.when(my == 0)
        def _():
            kvb[0] = k_ref[0].astype(jnp.bfloat16)
            kvb[1] = v_ref[0].astype(jnp.bfloat16)

        @pl.when(my == 1)
        def _():
            kvb[0, 0:WIN] = k_ref[0, 0:WIN].astype(jnp.bfloat16)
            kvb[1, 0:WIN] = v_ref[0, 0:WIN].astype(jnp.bfloat16)

        bar = pltpu.get_barrier_semaphore()
        for off in range(1, N_DEV):
            pl.semaphore_signal(
                bar, inc=1, device_id=((my + off) % N_DEV,),
                device_id_type=pl.DeviceIdType.MESH)
        pl.semaphore_wait(bar, N_DEV - 1)

        def rc(src, dst, ss, rs, tgt):
            return pltpu.make_async_remote_copy(
                src_ref=src, dst_ref=dst,
                send_sem=ksend.at[ss], recv_sem=krecv.at[rs],
                device_id=(tgt,), device_id_type=pl.DeviceIdType.MESH)

        d0_sends = [
            ((896, 384), (512, 384), 0, 0, 1),
            ((384, 512), (0, 512), 1, 1, 1),
            ((1280, 128), (NK, 128), 2, 0, 3),
            ((1408, 256), (0, 256), 3, 1, 3),
            ((1664, 384), (256, 384), 4, 2, 3),
        ]

        def rwait(dst, rs):
            rc(dst, dst, 0, rs, 0).wait_recv()

        def d0_desc(i):
            (s0, sn), (t0, tn), ss, rs, tgt = d0_sends[i]
            return rc(kvb.at[:, pl.ds(s0, sn)], kvw.at[:, pl.ds(t0, tn)],
                      ss, rs, tgt)

        @pl.when(my == 0)
        def _():
            for i in range(len(d0_sends)):
                d0_desc(i).start()
            kvw[:, 0:NK] = kvb[:, 0:NK]

        d1_halo = lambda: rc(kvb.at[:, pl.ds(0, WIN)],
                             kvw.at[:, pl.ds(NK - WIN, WIN)], 0, 3, 3)
        d1_fwd = lambda: rc(kvw.at[:, pl.ds(512, 384)],
                            kvw.at[:, pl.ds(0, 384)], 1, 1, 2)
        d3_fwd1 = lambda: rc(kvw.at[:, pl.ds(NK, WIN)],
                             kvw.at[:, pl.ds(384, WIN)], 0, 2, 2)
        d3_fwd2 = lambda: rc(kvw.at[:, pl.ds(0, 256)],
                             kvw.at[:, pl.ds(512, 256)], 1, 3, 2)

        @pl.when(my == 1)
        def _():
            d1_halo().start()
            rwait(kvw.at[:, pl.ds(512, 384)], 0)
            d1_fwd().start()
            rwait(kvw.at[:, pl.ds(0, 512)], 1)

        @pl.when(my == 3)
        def _():
            rwait(kvw.at[:, pl.ds(NK, WIN)], 0)
            rwait(kvw.at[:, pl.ds(0, 256)], 1)
            d3_fwd1().start()
            d3_fwd2().start()
            rwait(kvw.at[:, pl.ds(256, 384)], 2)
            rwait(kvw.at[:, pl.ds(NK - WIN, WIN)], 3)

        @pl.when(my == 2)
        def _():
            rwait(kvw.at[:, pl.ds(0, 384)], 1)
            rwait(kvw.at[:, pl.ds(384, WIN)], 2)
            rwait(kvw.at[:, pl.ds(512, 256)], 3)

        xcopy.wait()
        xb = xloc[...].astype(jnp.bfloat16)
        q = jnp.dot(xb, wq_ref[...].astype(jnp.bfloat16),
                    preferred_element_type=jnp.float32)

        base = jnp.maximum(my * QB - WIN, 0)
        qi = my * QB + lax.broadcasted_iota(jnp.int32, (QB, NK), 0)
        kj = base + lax.broadcasted_iota(jnp.int32, (QB, NK), 1)
        band = jnp.abs(qi - kj) <= WIN

        ctxs = []
        for h in range(HQ):
            qh = q[:, h * DH:(h + 1) * DH].astype(jnp.bfloat16)
            kh = kvw[0, 0:NK, h, :]
            s = lax.dot_general(qh, kh, (((1,), (1,)), ((), ())),
                                preferred_element_type=jnp.float32) * SCALE
            s = jnp.where(band, s, NEG)
            m = jnp.max(s, axis=1, keepdims=True)
            w = jnp.exp(s - m)
            l = jnp.sum(w, axis=1, keepdims=True)
            p = (w / l).astype(jnp.bfloat16)
            vh = kvw[1, 0:NK, h, :]
            ctxs.append(lax.dot_general(p, vh, (((1,), (0,)), ((), ())),
                                        preferred_element_type=jnp.float32))
        ctx = jnp.concatenate(ctxs, axis=1)
        ob = jnp.dot(ctx.astype(jnp.bfloat16),
                     wo_ref[...].astype(jnp.bfloat16),
                     preferred_element_type=jnp.float32)
        out_ref[0, pl.ds(my * QB, QB), :] = ob.astype(jnp.bfloat16)

        my_sl = pl.ds(my * QB, QB)
        s_l = pltpu.make_async_remote_copy(
            src_ref=out_ref.at[0, my_sl, :], dst_ref=out_ref.at[0, my_sl, :],
            send_sem=agsend.at[0], recv_sem=agrecv.at[0],
            device_id=(left,), device_id_type=pl.DeviceIdType.MESH)
        s_r = pltpu.make_async_remote_copy(
            src_ref=out_ref.at[0, my_sl, :], dst_ref=out_ref.at[0, my_sl, :],
            send_sem=agsend.at[1], recv_sem=agrecv.at[1],
            device_id=(right,), device_id_type=pl.DeviceIdType.MESH)
        s_l.start()
        s_r.start()
        s_l.wait()
        s_r.wait()
        f_r = pltpu.make_async_remote_copy(
            src_ref=out_ref.at[0, pl.ds(left * QB, QB // 2), :],
            dst_ref=out_ref.at[0, pl.ds(left * QB, QB // 2), :],
            send_sem=agsend.at[2], recv_sem=agrecv.at[2],
            device_id=(right,), device_id_type=pl.DeviceIdType.MESH)
        f_l = pltpu.make_async_remote_copy(
            src_ref=out_ref.at[0, pl.ds(right * QB + QB // 2, QB // 2), :],
            dst_ref=out_ref.at[0, pl.ds(right * QB + QB // 2, QB // 2), :],
            send_sem=agsend.at[3], recv_sem=agrecv.at[3],
            device_id=(left,), device_id_type=pl.DeviceIdType.MESH)
        f_r.start()
        f_l.start()
        f_r.wait()
        f_l.wait()

        @pl.when(my == 0)
        def _():
            for i in range(len(d0_sends)):
                d0_desc(i).wait_send()

        @pl.when(my == 1)
        def _():
            d1_halo().wait_send()
            d1_fwd().wait_send()

        @pl.when(my == 3)
        def _():
            d3_fwd1().wait_send()
            d3_fwd2().wait_send()

    return pl.pallas_call(
        body,
        out_shape=jax.ShapeDtypeStruct((1, SQ, D), jnp.bfloat16),
        in_specs=[
            pl.BlockSpec(memory_space=pl.ANY),
            pl.BlockSpec(memory_space=pltpu.VMEM),
            pl.BlockSpec(memory_space=pltpu.VMEM),
            pl.BlockSpec(memory_space=pltpu.VMEM),
            pl.BlockSpec(memory_space=pltpu.VMEM),
        ],
        out_specs=pl.BlockSpec(memory_space=pltpu.VMEM),
        scratch_shapes=[
            pltpu.VMEM((QB, D), jnp.float32),
            pltpu.VMEM((2, 2048, HQ, DH), jnp.bfloat16),
            pltpu.VMEM((2, NK + WIN, HQ, DH), jnp.bfloat16),
            pltpu.SemaphoreType.DMA,
            pltpu.SemaphoreType.DMA((5,)),
            pltpu.SemaphoreType.DMA((4,)),
            pltpu.SemaphoreType.DMA((4,)),
            pltpu.SemaphoreType.DMA((4,)),
        ],
        compiler_params=pltpu.CompilerParams(
            collective_id=0, vmem_limit_bytes=60 * 1024 * 1024),
    )(x, Wq, K_ext, V_ext, Wo)
